# baseline (speedup 1.0000x reference)
import jax
import jax.numpy as jnp
from jax import lax
from jax.experimental import pallas as pl
from jax.experimental.pallas import tpu as pltpu
from jax.experimental.pallas import tpu_sc as plsc

N_ENT = 10000
N_REL = 10000
N_PAD = 10240
D = 128
BN_EPS = 1e-5

E_TOTAL = 160000
E_PAD = 163840
N_TILES = 16
EPT = E_PAD // N_TILES
K = 32
CPT = EPT // K
SUPER = 8
NSUPER = CPT // SUPER
ROWS_E = E_PAD // K
ROWS_PER_TILE = 624
FIN_CHUNK = 16
SCAL_ROWS = 80
NODES_PER_TILE = N_PAD // N_TILES

TC_BLK = 1024


def _precompute_body(ent_ref, rel_ref, Wf_ref, w2_ref, b2_ref, bc_ref,
                     a0tab_ref, a1tab_ref, a2tab_ref, a0_ref, a1_ref, a2_ref):
    x = ent_ref[...]
    n = jnp.sqrt(jnp.sum(x * x, axis=1, keepdims=True))
    x = x * jnp.where(n > 1.0, 1.0 / (n + 1e-7), 1.0)
    y = rel_ref[...]
    m = jnp.sqrt(jnp.sum(y * y, axis=1, keepdims=True))
    y = y * jnp.where(m > 1.0, 1.0 / (m + 1e-7), 1.0)
    W = Wf_ref[...]
    dn = (((1,), (1,)), ((), ()))
    halfb = 0.5 * bc_ref[...]
    A0 = lax.dot_general(x, W[:, 0:D], dn, preferred_element_type=jnp.float32) + halfb
    A1 = lax.dot_general(x, W[:, D:2 * D], dn, preferred_element_type=jnp.float32) + halfb
    A2 = lax.dot_general(y, W[:, 2 * D:3 * D], dn, preferred_element_type=jnp.float32)
    w2 = w2_ref[...]
    b2 = b2_ref[0:1, 0:1]
    a0_ref[...] = lax.dot_general(w2, A0, dn, preferred_element_type=jnp.float32) + b2
    a1_ref[...] = lax.dot_general(w2, A1, dn, preferred_element_type=jnp.float32)
    a2_ref[...] = lax.dot_general(w2, A2, dn, preferred_element_type=jnp.float32)
    a0tab_ref[...] = A0
    a1tab_ref[...] = A1
    a2tab_ref[...] = A2


def _tc_precompute(ent_p, rel_p, Wf, w2, b2_arr, bc_arr):
    grid = (N_PAD // TC_BLK,)
    return pl.pallas_call(
        _precompute_body,
        grid=grid,
        in_specs=[
            pl.BlockSpec((TC_BLK, D), lambda i: (i, 0)),
            pl.BlockSpec((TC_BLK, D), lambda i: (i, 0)),
            pl.BlockSpec((D, 3 * D), lambda i: (0, 0)),
            pl.BlockSpec((1, D), lambda i: (0, 0)),
            pl.BlockSpec((1, D), lambda i: (0, 0)),
            pl.BlockSpec((1, D), lambda i: (0, 0)),
        ],
        out_specs=[
            pl.BlockSpec((TC_BLK, D), lambda i: (i, 0)),
            pl.BlockSpec((TC_BLK, D), lambda i: (i, 0)),
            pl.BlockSpec((TC_BLK, D), lambda i: (i, 0)),
            pl.BlockSpec((1, TC_BLK), lambda i: (0, i)),
            pl.BlockSpec((1, TC_BLK), lambda i: (0, i)),
            pl.BlockSpec((1, TC_BLK), lambda i: (0, i)),
        ],
        out_shape=[
            jax.ShapeDtypeStruct((N_PAD, D), jnp.float32),
            jax.ShapeDtypeStruct((N_PAD, D), jnp.float32),
            jax.ShapeDtypeStruct((N_PAD, D), jnp.float32),
            jax.ShapeDtypeStruct((1, N_PAD), jnp.float32),
            jax.ShapeDtypeStruct((1, N_PAD), jnp.float32),
            jax.ShapeDtypeStruct((1, N_PAD), jnp.float32),
        ],
    )(ent_p, rel_p, Wf, w2, b2_arr, bc_arr)


def _weights_body(a0_hbm, a1_hbm, a2_hbm, t0_hbm, t1_hbm, t2_hbm,
                  wf_hbm, wb_hbm, den8_hbm,
                  scal_acc, scal_acc2, t0f, t1f, t2f, wff, wbf, a0t, a1t, a2t,
                  ebs_l, ebs_l2, iden, zb16, d8b):
    cid = lax.axis_index("c")
    sid = lax.axis_index("s")
    is_ent = cid == 0
    flag = cid.astype(jnp.float32)
    base = sid * EPT

    pltpu.sync_copy(a0_hbm, a0t)
    pltpu.sync_copy(a1_hbm, a1t)
    pltpu.sync_copy(a2_hbm, a2t)
    pltpu.sync_copy(t0_hbm.at[pl.ds(base, EPT)], t0f.at[pl.ds(0, EPT)])
    pltpu.sync_copy(t1_hbm.at[pl.ds(base, EPT)], t1f.at[pl.ds(0, EPT)])
    pltpu.sync_copy(t2_hbm.at[pl.ds(base, EPT)], t2f.at[pl.ds(0, EPT)])

    iota16 = lax.iota(jnp.int32, 16)

    def zscal(g, c):
        iden[pl.ds(g * 16, 16)] = iota16 + g * 16
        return c
    lax.fori_loop(0, SCAL_ROWS // 16, zscal, 0)

    def zscal2(g, c):
        for j in range(D // 16):
            ebs_l[g, pl.ds(j * 16, 16)] = jnp.zeros((16,), jnp.float32)
            ebs_l2[g, pl.ds(j * 16, 16)] = jnp.zeros((16,), jnp.float32)
        return c
    lax.fori_loop(0, SCAL_ROWS, zscal2, 0)

    @pl.when(sid == 0)
    def _():
        def zr(i, c):
            for j in range(D // 16):
                zb16[i, pl.ds(j * 16, 16)] = jnp.zeros((16,), jnp.float32)
            return c
        lax.fori_loop(0, FIN_CHUNK, zr, 0)

        def zs(k, c):
            pltpu.sync_copy(zb16, scal_acc.at[pl.ds(k * FIN_CHUNK, FIN_CHUNK)])
            pltpu.sync_copy(zb16, scal_acc2.at[pl.ds(k * FIN_CHUNK, FIN_CHUNK)])
            return c
        lax.fori_loop(0, SCAL_ROWS // FIN_CHUNK, zs, 0)
    plsc.subcore_barrier()

    z16 = jnp.zeros((16,), jnp.int32)
    elim = jnp.full((16,), E_TOTAL, jnp.int32)

    def wstage(g, c):
        tv0 = t0f[pl.ds(g * 16, 16)]
        tv1 = t1f[pl.ds(g * 16, 16)]
        tv2 = t2f[pl.ds(g * 16, 16)]
        a0u = plsc.load_gather(a0t, [z16, tv0])
        a1u = plsc.load_gather(a1t, [z16, tv0])
        a0v = plsc.load_gather(a0t, [z16, tv1])
        a1v = plsc.load_gather(a1t, [z16, tv1])
        a2r = plsc.load_gather(a2t, [z16, tv2])
        zf = a0u + a1v + a2r
        zb = a0v + a1u - a2r
        gmask = ((iota16 + (base + g * 16)) < elim).astype(jnp.float32)
        wf = jnp.exp(jnp.minimum(-zf, -0.01 * zf)) * gmask
        wb = jnp.exp(jnp.minimum(-zb, -0.01 * zb)) * gmask
        wff[pl.ds(g * 16, 16)] = wf
        wbf[pl.ds(g * 16, 16)] = wb
        return c
    lax.fori_loop(0, EPT // 16, wstage, 0)

    @pl.when(is_ent)
    def _():
        pltpu.sync_copy(wff.at[pl.ds(0, EPT)], wf_hbm.at[pl.ds(base, EPT)])
        pltpu.sync_copy(wbf.at[pl.ds(0, EPT)], wb_hbm.at[pl.ds(base, EPT)])

    def acc_body(e, c):
        wf = wff[pl.ds(e, 16)][0]
        wb = wbf[pl.ds(e, 16)][0]
        t0s = t0f[pl.ds(e, 16)][0]
        t1s = t1f[pl.ds(e, 16)][0]
        t2s = t2f[pl.ds(e, 16)][0]
        m = jnp.where(base + e < E_TOTAL, jnp.float32(1.0), jnp.float32(0.0))
        na = t0s + (t2s - t0s) * cid
        ra = na >> 7
        ca = na & 112
        la = na & 15
        oh = (iota16 == la).astype(jnp.float32)
        da = wf + flag * (m - wf)
        ebs_l[ra, pl.ds(ca, 16)] = ebs_l[ra, pl.ds(ca, 16)] + oh * da

        @pl.when(is_ent)
        def _():
            rb = t1s >> 7
            cb2 = t1s & 112
            lb = t1s & 15
            ohb = (iota16 == lb).astype(jnp.float32) * wb
            ebs_l[rb, pl.ds(cb2, 16)] = ebs_l[rb, pl.ds(cb2, 16)] + ohb

        @pl.when(jnp.logical_not(is_ent))
        def _():
            ebs_l2[ra, pl.ds(ca, 16)] = ebs_l2[ra, pl.ds(ca, 16)] + oh * (wf + wb)
        return c
    lax.fori_loop(0, EPT, acc_body, 0)

    pltpu.sync_copy(ebs_l, scal_acc.at[iden], add=True)
    pltpu.sync_copy(ebs_l2, scal_acc2.at[iden], add=True)
    plsc.subcore_barrier()

    pltpu.sync_copy(scal_acc, ebs_l)
    pltpu.sync_copy(scal_acc2, ebs_l2)
    nbase = sid * NODES_PER_TILE
    f16 = iota16.astype(jnp.float32)
    mA0 = ((iota16 >> 2) == 0).astype(jnp.float32)
    mB0 = ((iota16 >> 2) == 1).astype(jnp.float32)
    mA1 = ((iota16 >> 2) == 2).astype(jnp.float32)
    mB1 = ((iota16 >> 2) == 3).astype(jnp.float32)
    del f16

    def expand(g, c):
        node0 = nbase + g * 16
        dr = node0 >> 7
        dc = node0 & 112
        dvA = ebs_l[dr, pl.ds(dc, 16)]
        dvB0 = ebs_l2[dr, pl.ds(dc, 16)]
        dvB = dvB0 + (1.0 - flag) * (dvA - dvB0)
        for h in range(8):
            pair = (mA0 * dvA[2 * h] + mB0 * dvB[2 * h]
                    + mA1 * dvA[2 * h + 1] + mB1 * dvB[2 * h + 1])
            d8b[pl.ds(g * 128 + h * 16, 16)] = pair
        return c
    lax.fori_loop(0, NODES_PER_TILE // 16, expand, 0)
    pltpu.sync_copy(d8b, den8_hbm.at[cid, pl.ds(nbase * 8, NODES_PER_TILE * 8)])


def _sc_weights(a0_t, a1_t, a2_t, t0, t1, t2):
    mesh = plsc.VectorSubcoreMesh(core_axis_name="c", subcore_axis_name="s")
    f = pl.kernel(
        _weights_body,
        out_type=(jax.ShapeDtypeStruct((E_PAD,), jnp.float32),
                  jax.ShapeDtypeStruct((E_PAD,), jnp.float32),
                  jax.ShapeDtypeStruct((2, N_PAD * 8), jnp.float32)),
        mesh=mesh,
        compiler_params=pltpu.CompilerParams(needs_layout_passes=False),
        scratch_types=[
            pltpu.VMEM_SHARED((SCAL_ROWS, D), jnp.float32),
            pltpu.VMEM_SHARED((SCAL_ROWS, D), jnp.float32),
            pltpu.VMEM((EPT + 16,), jnp.int32),
            pltpu.VMEM((EPT + 16,), jnp.int32),
            pltpu.VMEM((EPT + 16,), jnp.int32),
            pltpu.VMEM((EPT + 16,), jnp.float32),
            pltpu.VMEM((EPT + 16,), jnp.float32),
            pltpu.VMEM((1, N_PAD), jnp.float32),
            pltpu.VMEM((1, N_PAD), jnp.float32),
            pltpu.VMEM((1, N_PAD), jnp.float32),
            pltpu.VMEM((SCAL_ROWS, D), jnp.float32),
            pltpu.VMEM((SCAL_ROWS, D), jnp.float32),
            pltpu.VMEM((SCAL_ROWS,), jnp.int32),
            pltpu.VMEM((FIN_CHUNK, D), jnp.float32),
            pltpu.VMEM((NODES_PER_TILE * 8,), jnp.float32),
        ],
    )
    return f(a0_t, a1_t, a2_t, t0, t1, t2)


def _rows_body(T_hbm, gidx_hbm, sidx_hbm, wfr_hbm, wbr_hbm, acc3_hbm,
               acc, gst1, gst2, gst3, gst4, ssta, sstb, wst, wbst,
               b1a, b2a, b3a, b4a, b1b, b2b, b3b, b4b,
               bo1a, bo2a, bo1b, bo2b,
               sem_st, sem_g, sem_s):
    cid = lax.axis_index("c")
    sid = lax.axis_index("s")
    is_ent = cid == 0
    is_rel = jnp.logical_not(is_ent)
    flag = cid.astype(jnp.float32)

    n_fin = jnp.where(sid == N_TILES - 1, 40, 39)

    def zb4(i, c):
        for j in range(D // 32):
            b4a[i, pl.ds(j * 16, 16)] = jnp.zeros((16,), jnp.int32)
            b4b[i, pl.ds(j * 16, 16)] = jnp.zeros((16,), jnp.int32)
        return c
    lax.fori_loop(0, K, zb4, 0)

    def zrow(i, c):
        for j in range(D // 16):
            bo1a[i, pl.ds(j * 16, 16)] = jnp.zeros((16,), jnp.float32)
        return c
    lax.fori_loop(0, FIN_CHUNK, zrow, 0)

    def zcopy(k, c):
        pltpu.sync_copy(bo1a.at[pl.ds(0, FIN_CHUNK)],
                        acc.at[pl.ds(sid * ROWS_PER_TILE + k * FIN_CHUNK, FIN_CHUNK)])
        return c
    lax.fori_loop(0, n_fin, zcopy, 0)
    plsc.subcore_barrier()

    row_base = sid * CPT
    bufs = ((b1a, b2a, b3a, b4a), (b1b, b2b, b3b, b4b))
    obufs = ((bo1a, bo2a), (bo1b, bo2b))

    def fire_gathers(k, par):
        c1 = pltpu.async_copy(T_hbm.at[gst1.at[k]], bufs[par][0], sem_g)
        c2 = pltpu.async_copy(T_hbm.at[gst2.at[k]], bufs[par][1], sem_g)
        c3 = pltpu.async_copy(T_hbm.at[gst3.at[k]], bufs[par][2], sem_g)
        cs = [c1, c2, c3]

        @pl.when(is_rel)
        def _():
            cs.append(pltpu.async_copy(T_hbm.at[gst4.at[k]], bufs[par][3], sem_g))
        return cs

    def wait_gathers(cs):
        cs[0].wait()
        cs[1].wait()
        cs[2].wait()

        @pl.when(is_rel)
        def _():
            cs[3].wait()

    def compute(k, par):
        c1, c2, c3, c4 = bufs[par]
        o1, o2 = obufs[par]

        def grp(g, c):
            wfv = wst[k, pl.ds(g * 16, 16)]
            wbv = wbst[k, pl.ds(g * 16, 16)]
            for l in range(16):
                e = g * 16 + l
                wf = wfv[l]
                wb = wbv[l]
                for jj in range(D // 32):
                    sb = pl.ds(jj * 16, 16)
                    x1 = plsc.unpack(plsc.bitcast(c1[e, sb], jnp.bfloat16),
                                     format=plsc.PackFormat.INTERLEAVED)
                    x2 = plsc.unpack(plsc.bitcast(c2[e, sb], jnp.bfloat16),
                                     format=plsc.PackFormat.INTERLEAVED)
                    x3 = plsc.unpack(plsc.bitcast(c3[e, sb], jnp.bfloat16),
                                     format=plsc.PackFormat.INTERLEAVED)
                    x4 = plsc.unpack(plsc.bitcast(c4[e, sb], jnp.bfloat16),
                                     format=plsc.PackFormat.INTERLEAVED)
                    for h in range(2):
                        sl = pl.ds((2 * jj + h) * 16, 16)
                        ent1 = wb * (x1[h] - x3[h])
                        rel1 = wf * (x1[h] + x2[h]) - wb * (x3[h] + x4[h])
                        o2[e, sl] = wf * (x2[h] + x3[h])
                        o1[e, sl] = ent1 + flag * (rel1 - ent1)
            return c
        lax.fori_loop(0, K // 16, grp, 0)

    def fire_scatters(k, par):
        o1, o2 = obufs[par]

        @pl.when(is_ent)
        def _():
            pltpu.async_copy(o2, acc.at[ssta.at[k]], sem_s, add=True)
            pltpu.async_copy(o1, acc.at[sstb.at[k]], sem_s, add=True)

        @pl.when(is_rel)
        def _():
            pltpu.async_copy(o1, acc.at[ssta.at[k]], sem_s, add=True)

    def drain_scatters(par):
        @pl.when(is_ent)
        def _():
            pltpu.make_async_copy(acc3_hbm.at[0, pl.ds(0, K)], obufs[par][0], sem_s).wait()
            pltpu.make_async_copy(acc3_hbm.at[0, pl.ds(0, K)], obufs[par][1], sem_s).wait()

        @pl.when(is_rel)
        def _():
            pltpu.make_async_copy(acc3_hbm.at[0, pl.ds(0, K)], obufs[par][0], sem_s).wait()

    NPAIRS = CPT // 2

    def pair_body(p, carry):
        @pl.when(p % (SUPER // 2) == 0)
        def _():
            rb = row_base + (p // (SUPER // 2)) * SUPER
            s = [pltpu.async_copy(gidx_hbm.at[cid, 0, pl.ds(rb, SUPER)], gst1, sem_st),
                 pltpu.async_copy(gidx_hbm.at[cid, 1, pl.ds(rb, SUPER)], gst2, sem_st),
                 pltpu.async_copy(gidx_hbm.at[cid, 2, pl.ds(rb, SUPER)], gst3, sem_st),
                 pltpu.async_copy(gidx_hbm.at[cid, 3, pl.ds(rb, SUPER)], gst4, sem_st),
                 pltpu.async_copy(sidx_hbm.at[cid, 0, pl.ds(rb, SUPER)], ssta, sem_st),
                 pltpu.async_copy(sidx_hbm.at[cid, 1, pl.ds(rb, SUPER)], sstb, sem_st),
                 pltpu.async_copy(wfr_hbm.at[cid, pl.ds(rb, SUPER)], wst, sem_st),
                 pltpu.async_copy(wbr_hbm.at[cid, pl.ds(rb, SUPER)], wbst, sem_st)]
            for c in s:
                c.wait()

        kA = (2 * p) % SUPER
        kB = kA + 1

        @pl.when(p != 0)
        def _():
            drain_scatters(0)
            drain_scatters(1)

        ga = fire_gathers(kA, 0)
        gb = fire_gathers(kB, 1)

        wait_gathers(ga)
        compute(kA, 0)
        fire_scatters(kA, 0)

        wait_gathers(gb)
        compute(kB, 1)
        fire_scatters(kB, 1)
        return carry

    lax.fori_loop(0, NPAIRS, pair_body, 0)
    drain_scatters(0)
    drain_scatters(1)
    plsc.subcore_barrier()

    abase = sid * ROWS_PER_TILE
    pltpu.sync_copy(acc.at[pl.ds(abase, ROWS_PER_TILE)],
                    acc3_hbm.at[cid, pl.ds(abase, ROWS_PER_TILE)])

    @pl.when(sid == N_TILES - 1)
    def _():
        pltpu.sync_copy(acc.at[pl.ds(N_ENT - FIN_CHUNK, FIN_CHUNK)],
                        acc3_hbm.at[cid, pl.ds(N_ENT - FIN_CHUNK, FIN_CHUNK)])


def _sc_rows(T, gidx, sidx, wfr, wbr):
    mesh = plsc.VectorSubcoreMesh(core_axis_name="c", subcore_axis_name="s")
    f = pl.kernel(
        _rows_body,
        out_type=jax.ShapeDtypeStruct((2, N_ENT, D), jnp.float32),
        mesh=mesh,
        compiler_params=pltpu.CompilerParams(needs_layout_passes=False, use_tc_tiling_on_sc=False),
        scratch_types=[
            pltpu.VMEM_SHARED((N_ENT, D), jnp.float32),
            pltpu.VMEM((SUPER, K), jnp.int32),
            pltpu.VMEM((SUPER, K), jnp.int32),
            pltpu.VMEM((SUPER, K), jnp.int32),
            pltpu.VMEM((SUPER, K), jnp.int32),
            pltpu.VMEM((SUPER, K), jnp.int32),
            pltpu.VMEM((SUPER, K), jnp.int32),
            pltpu.VMEM((SUPER, K), jnp.float32),
            pltpu.VMEM((SUPER, K), jnp.float32),
            pltpu.VMEM((K, D // 2), jnp.int32),
            pltpu.VMEM((K, D // 2), jnp.int32),
            pltpu.VMEM((K, D // 2), jnp.int32),
            pltpu.VMEM((K, D // 2), jnp.int32),
            pltpu.VMEM((K, D // 2), jnp.int32),
            pltpu.VMEM((K, D // 2), jnp.int32),
            pltpu.VMEM((K, D // 2), jnp.int32),
            pltpu.VMEM((K, D // 2), jnp.int32),
            pltpu.VMEM((K, D), jnp.float32),
            pltpu.VMEM((K, D), jnp.float32),
            pltpu.VMEM((K, D), jnp.float32),
            pltpu.VMEM((K, D), jnp.float32),
            pltpu.SemaphoreType.DMA,
            pltpu.SemaphoreType.DMA,
            pltpu.SemaphoreType.DMA,
        ],
    )
    return f(T, gidx, sidx, wfr, wbr)


def _finalize_body(acc_ref, den_ref, diag_ref, out_ref):
    c = pl.program_id(0)
    num = acc_ref[0]
    den = den_ref[0][:, 0:1]
    scale = den_ref[0][:, 4:5]
    num = num + diag_ref[0] * scale
    den_e = jnp.where(den == 0.0, jnp.float32(1e-12), den)
    den_r = jnp.maximum(den, 1.0)
    den = jnp.where(c == 0, den_e, den_r)
    h = num / den
    out_ref[0] = jnp.where(h > 0.0, h, jnp.exp(h) - 1.0)


TC_BLK_F = 1000


def _tc_finalize(acc3, den8, diag):
    grid = (2, N_ENT // TC_BLK_F)
    return pl.pallas_call(
        _finalize_body,
        grid=grid,
        in_specs=[
            pl.BlockSpec((1, TC_BLK_F, D), lambda c, i: (c, i, 0)),
            pl.BlockSpec((1, TC_BLK_F, 8), lambda c, i: (c, i, 0)),
            pl.BlockSpec((1, TC_BLK_F, D), lambda c, i: (c, i, 0)),
        ],
        out_specs=pl.BlockSpec((1, TC_BLK_F, D), lambda c, i: (c, i, 0)),
        out_shape=jax.ShapeDtypeStruct((2, N_ENT, D), jnp.float32),
    )(acc3, den8, diag)


def kernel(triplets, ent_w, rel_w, W_a, b_a, W_a2, b_a2, g0, be0, g1, be1):
    inv = 1.0 / jnp.sqrt(jnp.float32(1.0 + BN_EPS))
    s0 = g0 * inv
    s1 = g1 * inv
    Wf = (s1[:, None] * W_a) * s0[None, :]
    bias_c = s1 * (W_a @ be0 + b_a) + be1
    b2_arr = jnp.full((1, D), b_a2[0], jnp.float32)
    bc_arr = bias_c.reshape(1, D)

    ent_p = jnp.pad(ent_w, ((0, N_PAD - N_ENT), (0, 0)))
    rel_p = jnp.pad(rel_w, ((0, N_PAD - N_REL), (0, 0)))

    a0tab, a1tab, a2tab, a0_t, a1_t, a2_t = _tc_precompute(
        ent_p, rel_p, Wf, W_a2, b2_arr, bc_arr)

    pad = E_PAD - E_TOTAL
    t0 = jnp.pad(triplets[:, 0], (0, pad))
    t1 = jnp.pad(triplets[:, 1], (0, pad))
    t2 = jnp.pad(triplets[:, 2], (0, pad))

    wf, wb, den8f = _sc_weights(a0_t, a1_t, a2_t, t0, t1, t2)
    den8 = den8f.reshape(2, N_PAD, 8)

    import numpy as _np
    perm32 = _np.array([x for i in range(16) for x in (i, 16 + i)], _np.int32)
    perm = _np.concatenate([32 * b + perm32 for b in range(D // 32)])
    Tb = jnp.concatenate([a0tab, a1tab, a2tab], axis=0)[:, perm].astype(jnp.bfloat16)
    T = lax.bitcast_convert_type(Tb.reshape(3 * N_PAD, D // 2, 2), jnp.int32)

    gidx = jnp.stack([
        jnp.stack([t0 + N_PAD, t1 + N_PAD, t2 + 2 * N_PAD, t0 + N_PAD]),
        jnp.stack([t0, t1 + N_PAD, t1, t0 + N_PAD]),
    ]).reshape(2, 4, ROWS_E, K)
    sidx = jnp.stack([
        jnp.stack([t0, t1]),
        jnp.stack([t2, t2]),
    ]).reshape(2, 2, ROWS_E, K)
    wfr = jnp.broadcast_to(wf.reshape(1, ROWS_E, K), (2, ROWS_E, K))
    wbr = jnp.broadcast_to(wb.reshape(1, ROWS_E, K), (2, ROWS_E, K))

    acc3 = _sc_rows(T, gidx, sidx, wfr, wbr)

    diag = jnp.stack([a0tab[:N_ENT], a2tab[:N_ENT]])

    h3 = _tc_finalize(acc3, den8[:, :N_ENT, :], diag)
    return h3[0], h3[1]


# speedup vs baseline: 1.6130x; 1.6130x over previous
import jax
import jax.numpy as jnp
from jax import lax
from jax.experimental import pallas as pl
from jax.experimental.pallas import tpu as pltpu
from jax.experimental.pallas import tpu_sc as plsc

N_ENT = 10000
N_REL = 10000
N_PAD = 10240
D = 128
BN_EPS = 1e-5

E_TOTAL = 160000
E_PAD = 163840
N_TILES = 16
EPT = E_PAD // N_TILES
K = 32
CPT = EPT // K
SUPER = 8
NSUPER = CPT // SUPER
ROWS_E = E_PAD // K
ROWS_PER_TILE = 624
FIN_CHUNK = 16
SCAL_ROWS = 80
NODES_PER_TILE = N_PAD // N_TILES

TC_BLK = 1024


def _precompute_body(ent_ref, rel_ref, Wf_ref, w2_ref, b2_ref, bc_ref,
                     a0tab_ref, a1tab_ref, a2tab_ref, a0_ref, a1_ref, a2_ref):
    x = ent_ref[...]
    n = jnp.sqrt(jnp.sum(x * x, axis=1, keepdims=True))
    x = x * jnp.where(n > 1.0, 1.0 / (n + 1e-7), 1.0)
    y = rel_ref[...]
    m = jnp.sqrt(jnp.sum(y * y, axis=1, keepdims=True))
    y = y * jnp.where(m > 1.0, 1.0 / (m + 1e-7), 1.0)
    W = Wf_ref[...]
    dn = (((1,), (1,)), ((), ()))
    halfb = 0.5 * bc_ref[...]
    A0 = lax.dot_general(x, W[:, 0:D], dn, preferred_element_type=jnp.float32) + halfb
    A1 = lax.dot_general(x, W[:, D:2 * D], dn, preferred_element_type=jnp.float32) + halfb
    A2 = lax.dot_general(y, W[:, 2 * D:3 * D], dn, preferred_element_type=jnp.float32)
    w2 = w2_ref[...]
    b2 = b2_ref[0:1, 0:1]
    a0_ref[...] = lax.dot_general(w2, A0, dn, preferred_element_type=jnp.float32) + b2
    a1_ref[...] = lax.dot_general(w2, A1, dn, preferred_element_type=jnp.float32)
    a2_ref[...] = lax.dot_general(w2, A2, dn, preferred_element_type=jnp.float32)
    a0tab_ref[...] = A0
    a1tab_ref[...] = A1
    a2tab_ref[...] = A2


def _tc_precompute(ent_p, rel_p, Wf, w2, b2_arr, bc_arr):
    grid = (N_PAD // TC_BLK,)
    return pl.pallas_call(
        _precompute_body,
        grid=grid,
        in_specs=[
            pl.BlockSpec((TC_BLK, D), lambda i: (i, 0)),
            pl.BlockSpec((TC_BLK, D), lambda i: (i, 0)),
            pl.BlockSpec((D, 3 * D), lambda i: (0, 0)),
            pl.BlockSpec((1, D), lambda i: (0, 0)),
            pl.BlockSpec((1, D), lambda i: (0, 0)),
            pl.BlockSpec((1, D), lambda i: (0, 0)),
        ],
        out_specs=[
            pl.BlockSpec((TC_BLK, D), lambda i: (i, 0)),
            pl.BlockSpec((TC_BLK, D), lambda i: (i, 0)),
            pl.BlockSpec((TC_BLK, D), lambda i: (i, 0)),
            pl.BlockSpec((1, TC_BLK), lambda i: (0, i)),
            pl.BlockSpec((1, TC_BLK), lambda i: (0, i)),
            pl.BlockSpec((1, TC_BLK), lambda i: (0, i)),
        ],
        out_shape=[
            jax.ShapeDtypeStruct((N_PAD, D), jnp.float32),
            jax.ShapeDtypeStruct((N_PAD, D), jnp.float32),
            jax.ShapeDtypeStruct((N_PAD, D), jnp.float32),
            jax.ShapeDtypeStruct((1, N_PAD), jnp.float32),
            jax.ShapeDtypeStruct((1, N_PAD), jnp.float32),
            jax.ShapeDtypeStruct((1, N_PAD), jnp.float32),
        ],
    )(ent_p, rel_p, Wf, w2, b2_arr, bc_arr)


def _weights_body(a0_hbm, a1_hbm, a2_hbm, t0_hbm, t1_hbm, t2_hbm,
                  wf_hbm, wb_hbm, den8_hbm,
                  scal_acc, scal_acc2, t0f, t1f, t2f, wff, wbf, a0t, a1t, a2t,
                  ebs_l, ebs_l2, iden, zb16, d8b):
    cid = lax.axis_index("c")
    sid = lax.axis_index("s")
    is_ent = cid == 0
    flag = cid.astype(jnp.float32)
    base = sid * EPT

    pltpu.sync_copy(a0_hbm, a0t)
    pltpu.sync_copy(a1_hbm, a1t)
    pltpu.sync_copy(a2_hbm, a2t)
    pltpu.sync_copy(t0_hbm.at[pl.ds(base, EPT)], t0f.at[pl.ds(0, EPT)])
    pltpu.sync_copy(t1_hbm.at[pl.ds(base, EPT)], t1f.at[pl.ds(0, EPT)])
    pltpu.sync_copy(t2_hbm.at[pl.ds(base, EPT)], t2f.at[pl.ds(0, EPT)])

    iota16 = lax.iota(jnp.int32, 16)

    def zscal(g, c):
        iden[pl.ds(g * 16, 16)] = iota16 + g * 16
        return c
    lax.fori_loop(0, SCAL_ROWS // 16, zscal, 0)

    def zscal2(g, c):
        for j in range(D // 16):
            ebs_l[g, pl.ds(j * 16, 16)] = jnp.zeros((16,), jnp.float32)
            ebs_l2[g, pl.ds(j * 16, 16)] = jnp.zeros((16,), jnp.float32)
        return c
    lax.fori_loop(0, SCAL_ROWS, zscal2, 0)

    @pl.when(sid == 0)
    def _():
        def zr(i, c):
            for j in range(D // 16):
                zb16[i, pl.ds(j * 16, 16)] = jnp.zeros((16,), jnp.float32)
            return c
        lax.fori_loop(0, FIN_CHUNK, zr, 0)

        def zs(k, c):
            pltpu.sync_copy(zb16, scal_acc.at[pl.ds(k * FIN_CHUNK, FIN_CHUNK)])
            pltpu.sync_copy(zb16, scal_acc2.at[pl.ds(k * FIN_CHUNK, FIN_CHUNK)])
            return c
        lax.fori_loop(0, SCAL_ROWS // FIN_CHUNK, zs, 0)
    plsc.subcore_barrier()

    z16 = jnp.zeros((16,), jnp.int32)
    elim = jnp.full((16,), E_TOTAL, jnp.int32)

    def wstage(g, c):
        tv0 = t0f[pl.ds(g * 16, 16)]
        tv1 = t1f[pl.ds(g * 16, 16)]
        tv2 = t2f[pl.ds(g * 16, 16)]
        a0u = plsc.load_gather(a0t, [z16, tv0])
        a1u = plsc.load_gather(a1t, [z16, tv0])
        a0v = plsc.load_gather(a0t, [z16, tv1])
        a1v = plsc.load_gather(a1t, [z16, tv1])
        a2r = plsc.load_gather(a2t, [z16, tv2])
        zf = a0u + a1v + a2r
        zb = a0v + a1u - a2r
        gmask = ((iota16 + (base + g * 16)) < elim).astype(jnp.float32)
        wf = jnp.exp(jnp.minimum(-zf, -0.01 * zf)) * gmask
        wb = jnp.exp(jnp.minimum(-zb, -0.01 * zb)) * gmask
        wff[pl.ds(g * 16, 16)] = wf
        wbf[pl.ds(g * 16, 16)] = wb
        return c
    lax.fori_loop(0, EPT // 16, wstage, 0)

    @pl.when(is_ent)
    def _():
        pltpu.sync_copy(wff.at[pl.ds(0, EPT)], wf_hbm.at[pl.ds(base, EPT)])
        pltpu.sync_copy(wbf.at[pl.ds(0, EPT)], wb_hbm.at[pl.ds(base, EPT)])

    def acc_body(e, c):
        wf = wff[pl.ds(e, 16)][0]
        wb = wbf[pl.ds(e, 16)][0]
        t0s = t0f[pl.ds(e, 16)][0]
        t1s = t1f[pl.ds(e, 16)][0]
        t2s = t2f[pl.ds(e, 16)][0]
        m = jnp.where(base + e < E_TOTAL, jnp.float32(1.0), jnp.float32(0.0))
        na = t0s + (t2s - t0s) * cid
        ra = na >> 7
        ca = na & 112
        la = na & 15
        oh = (iota16 == la).astype(jnp.float32)
        da = wf + flag * (m - wf)
        ebs_l[ra, pl.ds(ca, 16)] = ebs_l[ra, pl.ds(ca, 16)] + oh * da

        @pl.when(is_ent)
        def _():
            rb = t1s >> 7
            cb2 = t1s & 112
            lb = t1s & 15
            ohb = (iota16 == lb).astype(jnp.float32) * wb
            ebs_l[rb, pl.ds(cb2, 16)] = ebs_l[rb, pl.ds(cb2, 16)] + ohb

        @pl.when(jnp.logical_not(is_ent))
        def _():
            ebs_l2[ra, pl.ds(ca, 16)] = ebs_l2[ra, pl.ds(ca, 16)] + oh * (wf + wb)
        return c
    lax.fori_loop(0, EPT, acc_body, 0)

    pltpu.sync_copy(ebs_l, scal_acc.at[iden], add=True)
    pltpu.sync_copy(ebs_l2, scal_acc2.at[iden], add=True)
    plsc.subcore_barrier()

    pltpu.sync_copy(scal_acc, ebs_l)
    pltpu.sync_copy(scal_acc2, ebs_l2)
    nbase = sid * NODES_PER_TILE
    f16 = iota16.astype(jnp.float32)
    mA0 = ((iota16 >> 2) == 0).astype(jnp.float32)
    mB0 = ((iota16 >> 2) == 1).astype(jnp.float32)
    mA1 = ((iota16 >> 2) == 2).astype(jnp.float32)
    mB1 = ((iota16 >> 2) == 3).astype(jnp.float32)
    del f16

    def expand(g, c):
        node0 = nbase + g * 16
        dr = node0 >> 7
        dc = node0 & 112
        dvA = ebs_l[dr, pl.ds(dc, 16)]
        dvB0 = ebs_l2[dr, pl.ds(dc, 16)]
        dvB = dvB0 + (1.0 - flag) * (dvA - dvB0)
        for h in range(8):
            pair = (mA0 * dvA[2 * h] + mB0 * dvB[2 * h]
                    + mA1 * dvA[2 * h + 1] + mB1 * dvB[2 * h + 1])
            d8b[pl.ds(g * 128 + h * 16, 16)] = pair
        return c
    lax.fori_loop(0, NODES_PER_TILE // 16, expand, 0)
    pltpu.sync_copy(d8b, den8_hbm.at[cid, pl.ds(nbase * 8, NODES_PER_TILE * 8)])


def _sc_weights(a0_t, a1_t, a2_t, t0, t1, t2):
    mesh = plsc.VectorSubcoreMesh(core_axis_name="c", subcore_axis_name="s")
    f = pl.kernel(
        _weights_body,
        out_type=(jax.ShapeDtypeStruct((E_PAD,), jnp.float32),
                  jax.ShapeDtypeStruct((E_PAD,), jnp.float32),
                  jax.ShapeDtypeStruct((2, N_PAD * 8), jnp.float32)),
        mesh=mesh,
        compiler_params=pltpu.CompilerParams(needs_layout_passes=False),
        scratch_types=[
            pltpu.VMEM_SHARED((SCAL_ROWS, D), jnp.float32),
            pltpu.VMEM_SHARED((SCAL_ROWS, D), jnp.float32),
            pltpu.VMEM((EPT + 16,), jnp.int32),
            pltpu.VMEM((EPT + 16,), jnp.int32),
            pltpu.VMEM((EPT + 16,), jnp.int32),
            pltpu.VMEM((EPT + 16,), jnp.float32),
            pltpu.VMEM((EPT + 16,), jnp.float32),
            pltpu.VMEM((1, N_PAD), jnp.float32),
            pltpu.VMEM((1, N_PAD), jnp.float32),
            pltpu.VMEM((1, N_PAD), jnp.float32),
            pltpu.VMEM((SCAL_ROWS, D), jnp.float32),
            pltpu.VMEM((SCAL_ROWS, D), jnp.float32),
            pltpu.VMEM((SCAL_ROWS,), jnp.int32),
            pltpu.VMEM((FIN_CHUNK, D), jnp.float32),
            pltpu.VMEM((NODES_PER_TILE * 8,), jnp.float32),
        ],
    )
    return f(a0_t, a1_t, a2_t, t0, t1, t2)


def _rows_body(T_hbm, gidx_hbm, sidx_hbm, wfr_hbm, wbr_hbm, acc3_hbm,
               acc, gst1, gst2, gst3, gst4, ssta, sstb, wst, wbst,
               b1a, b2a, b3a, b4a, b1b, b2b, b3b, b4b,
               sem_st, sem_g, sem_s):
    cid = lax.axis_index("c")
    sid = lax.axis_index("s")
    is_ent = cid == 0
    is_rel = jnp.logical_not(is_ent)
    flag = cid.astype(jnp.float32)

    n_fin = jnp.where(sid == N_TILES - 1, 40, 39)

    def zb4(i, c):
        for j in range(D // 16):
            b4a[i, pl.ds(j * 16, 16)] = jnp.zeros((16,), jnp.float32)
            b4b[i, pl.ds(j * 16, 16)] = jnp.zeros((16,), jnp.float32)
        return c
    lax.fori_loop(0, K, zb4, 0)

    def zrow(i, c):
        for j in range(D // 16):
            b1a[i, pl.ds(j * 16, 16)] = jnp.zeros((16,), jnp.float32)
        return c
    lax.fori_loop(0, FIN_CHUNK, zrow, 0)

    def zcopy(k, c):
        pltpu.sync_copy(b1a.at[pl.ds(0, FIN_CHUNK)],
                        acc.at[pl.ds(sid * ROWS_PER_TILE + k * FIN_CHUNK, FIN_CHUNK)])
        return c
    lax.fori_loop(0, n_fin, zcopy, 0)
    plsc.subcore_barrier()

    row_base = sid * CPT
    bufs = ((b1a, b2a, b3a, b4a), (b1b, b2b, b3b, b4b))

    def fire_gathers(k, par):
        c1 = pltpu.async_copy(T_hbm.at[gst1.at[k]], bufs[par][0], sem_g)
        c2 = pltpu.async_copy(T_hbm.at[gst2.at[k]], bufs[par][1], sem_g)
        c3 = pltpu.async_copy(T_hbm.at[gst3.at[k]], bufs[par][2], sem_g)
        cs = [c1, c2, c3]

        @pl.when(is_rel)
        def _():
            cs.append(pltpu.async_copy(T_hbm.at[gst4.at[k]], bufs[par][3], sem_g))
        return cs

    def wait_gathers(cs):
        cs[0].wait()
        cs[1].wait()
        cs[2].wait()

        @pl.when(is_rel)
        def _():
            cs[3].wait()

    def compute(k, par):
        c1, c2, c3, c4 = bufs[par]

        def grp(g, c):
            wfv = wst[k, pl.ds(g * 16, 16)]
            wbv = wbst[k, pl.ds(g * 16, 16)]
            for l in range(16):
                e = g * 16 + l
                wf = wfv[l]
                wb = wbv[l]
                for j in range(D // 16):
                    sl = pl.ds(j * 16, 16)
                    l1 = c1[e, sl]
                    l2 = c2[e, sl]
                    l3 = c3[e, sl]
                    l4 = c4[e, sl]
                    ent1 = wb * (l1 - l3)
                    rel1 = wf * (l1 + l2) - wb * (l3 + l4)
                    c2[e, sl] = wf * (l2 + l3)
                    c1[e, sl] = ent1 + flag * (rel1 - ent1)
            return c
        lax.fori_loop(0, K // 16, grp, 0)

    def fire_scatters(k, par):
        c1b, c2b = bufs[par][0], bufs[par][1]

        @pl.when(is_ent)
        def _():
            pltpu.async_copy(c2b, acc.at[ssta.at[k]], sem_s, add=True)
            pltpu.async_copy(c1b, acc.at[sstb.at[k]], sem_s, add=True)

        @pl.when(is_rel)
        def _():
            pltpu.async_copy(c1b, acc.at[ssta.at[k]], sem_s, add=True)

    def drain_scatters(par):
        @pl.when(is_ent)
        def _():
            pltpu.make_async_copy(T_hbm.at[pl.ds(0, K)], bufs[par][0], sem_s).wait()
            pltpu.make_async_copy(T_hbm.at[pl.ds(0, K)], bufs[par][1], sem_s).wait()

        @pl.when(is_rel)
        def _():
            pltpu.make_async_copy(T_hbm.at[pl.ds(0, K)], bufs[par][0], sem_s).wait()

    NPAIRS = CPT // 2

    def pair_body(p, carry):
        @pl.when(p % (SUPER // 2) == 0)
        def _():
            rb = row_base + (p // (SUPER // 2)) * SUPER
            s = [pltpu.async_copy(gidx_hbm.at[cid, 0, pl.ds(rb, SUPER)], gst1, sem_st),
                 pltpu.async_copy(gidx_hbm.at[cid, 1, pl.ds(rb, SUPER)], gst2, sem_st),
                 pltpu.async_copy(gidx_hbm.at[cid, 2, pl.ds(rb, SUPER)], gst3, sem_st),
                 pltpu.async_copy(gidx_hbm.at[cid, 3, pl.ds(rb, SUPER)], gst4, sem_st),
                 pltpu.async_copy(sidx_hbm.at[cid, 0, pl.ds(rb, SUPER)], ssta, sem_st),
                 pltpu.async_copy(sidx_hbm.at[cid, 1, pl.ds(rb, SUPER)], sstb, sem_st),
                 pltpu.async_copy(wfr_hbm.at[cid, pl.ds(rb, SUPER)], wst, sem_st),
                 pltpu.async_copy(wbr_hbm.at[cid, pl.ds(rb, SUPER)], wbst, sem_st)]
            for c in s:
                c.wait()

        kA = (2 * p) % SUPER
        kB = kA + 1

        @pl.when(p != 0)
        def _():
            drain_scatters(0)
            drain_scatters(1)

        ga = fire_gathers(kA, 0)
        gb = fire_gathers(kB, 1)

        wait_gathers(ga)
        compute(kA, 0)
        fire_scatters(kA, 0)

        wait_gathers(gb)
        compute(kB, 1)
        fire_scatters(kB, 1)
        return carry

    lax.fori_loop(0, NPAIRS, pair_body, 0)
    drain_scatters(0)
    drain_scatters(1)
    plsc.subcore_barrier()

    abase = sid * ROWS_PER_TILE
    pltpu.sync_copy(acc.at[pl.ds(abase, ROWS_PER_TILE)],
                    acc3_hbm.at[cid, pl.ds(abase, ROWS_PER_TILE)])

    @pl.when(sid == N_TILES - 1)
    def _():
        pltpu.sync_copy(acc.at[pl.ds(N_ENT - FIN_CHUNK, FIN_CHUNK)],
                        acc3_hbm.at[cid, pl.ds(N_ENT - FIN_CHUNK, FIN_CHUNK)])


def _sc_rows(T, gidx, sidx, wfr, wbr):
    mesh = plsc.VectorSubcoreMesh(core_axis_name="c", subcore_axis_name="s")
    f = pl.kernel(
        _rows_body,
        out_type=jax.ShapeDtypeStruct((2, N_ENT, D), jnp.float32),
        mesh=mesh,
        compiler_params=pltpu.CompilerParams(needs_layout_passes=False),
        scratch_types=[
            pltpu.VMEM_SHARED((N_ENT, D), jnp.float32),
            pltpu.VMEM((SUPER, K), jnp.int32),
            pltpu.VMEM((SUPER, K), jnp.int32),
            pltpu.VMEM((SUPER, K), jnp.int32),
            pltpu.VMEM((SUPER, K), jnp.int32),
            pltpu.VMEM((SUPER, K), jnp.int32),
            pltpu.VMEM((SUPER, K), jnp.int32),
            pltpu.VMEM((SUPER, K), jnp.float32),
            pltpu.VMEM((SUPER, K), jnp.float32),
            pltpu.VMEM((K, D), jnp.float32),
            pltpu.VMEM((K, D), jnp.float32),
            pltpu.VMEM((K, D), jnp.float32),
            pltpu.VMEM((K, D), jnp.float32),
            pltpu.VMEM((K, D), jnp.float32),
            pltpu.VMEM((K, D), jnp.float32),
            pltpu.VMEM((K, D), jnp.float32),
            pltpu.VMEM((K, D), jnp.float32),
            pltpu.SemaphoreType.DMA,
            pltpu.SemaphoreType.DMA,
            pltpu.SemaphoreType.DMA,
        ],
    )
    return f(T, gidx, sidx, wfr, wbr)


def _finalize_body(acc_ref, den_ref, diag_ref, out_ref):
    c = pl.program_id(0)
    num = acc_ref[0]
    den = den_ref[0][:, 0:1]
    scale = den_ref[0][:, 4:5]
    num = num + diag_ref[0] * scale
    den_e = jnp.where(den == 0.0, jnp.float32(1e-12), den)
    den_r = jnp.maximum(den, 1.0)
    den = jnp.where(c == 0, den_e, den_r)
    h = num / den
    out_ref[0] = jnp.where(h > 0.0, h, jnp.exp(h) - 1.0)


TC_BLK_F = 1000


def _tc_finalize(acc3, den8, diag):
    grid = (2, N_ENT // TC_BLK_F)
    return pl.pallas_call(
        _finalize_body,
        grid=grid,
        in_specs=[
            pl.BlockSpec((1, TC_BLK_F, D), lambda c, i: (c, i, 0)),
            pl.BlockSpec((1, TC_BLK_F, 8), lambda c, i: (c, i, 0)),
            pl.BlockSpec((1, TC_BLK_F, D), lambda c, i: (c, i, 0)),
        ],
        out_specs=pl.BlockSpec((1, TC_BLK_F, D), lambda c, i: (c, i, 0)),
        out_shape=jax.ShapeDtypeStruct((2, N_ENT, D), jnp.float32),
    )(acc3, den8, diag)


def kernel(triplets, ent_w, rel_w, W_a, b_a, W_a2, b_a2, g0, be0, g1, be1):
    inv = 1.0 / jnp.sqrt(jnp.float32(1.0 + BN_EPS))
    s0 = g0 * inv
    s1 = g1 * inv
    Wf = (s1[:, None] * W_a) * s0[None, :]
    bias_c = s1 * (W_a @ be0 + b_a) + be1
    b2_arr = jnp.full((1, D), b_a2[0], jnp.float32)
    bc_arr = bias_c.reshape(1, D)

    ent_p = jnp.pad(ent_w, ((0, N_PAD - N_ENT), (0, 0)))
    rel_p = jnp.pad(rel_w, ((0, N_PAD - N_REL), (0, 0)))

    a0tab, a1tab, a2tab, a0_t, a1_t, a2_t = _tc_precompute(
        ent_p, rel_p, Wf, W_a2, b2_arr, bc_arr)

    pad = E_PAD - E_TOTAL
    t0 = jnp.pad(triplets[:, 0], (0, pad))
    t1 = jnp.pad(triplets[:, 1], (0, pad))
    t2 = jnp.pad(triplets[:, 2], (0, pad))

    wf, wb, den8f = _sc_weights(a0_t, a1_t, a2_t, t0, t1, t2)
    den8 = den8f.reshape(2, N_PAD, 8)

    T = jnp.concatenate([a0tab, a1tab, a2tab], axis=0)

    gidx = jnp.stack([
        jnp.stack([t0 + N_PAD, t1 + N_PAD, t2 + 2 * N_PAD, t0 + N_PAD]),
        jnp.stack([t0, t1 + N_PAD, t1, t0 + N_PAD]),
    ]).reshape(2, 4, ROWS_E, K)
    sidx = jnp.stack([
        jnp.stack([t0, t1]),
        jnp.stack([t2, t2]),
    ]).reshape(2, 2, ROWS_E, K)
    wfr = jnp.broadcast_to(wf.reshape(1, ROWS_E, K), (2, ROWS_E, K))
    wbr = jnp.broadcast_to(wb.reshape(1, ROWS_E, K), (2, ROWS_E, K))

    acc3 = _sc_rows(T, gidx, sidx, wfr, wbr)

    diag = jnp.stack([a0tab[:N_ENT], a2tab[:N_ENT]])

    h3 = _tc_finalize(acc3, den8[:, :N_ENT, :], diag)
    return h3[0], h3[1]


# speedup vs baseline: 1.6289x; 1.0099x over previous
import jax
import jax.numpy as jnp
from jax import lax
from jax.experimental import pallas as pl
from jax.experimental.pallas import tpu as pltpu
from jax.experimental.pallas import tpu_sc as plsc

N_ENT = 10000
N_REL = 10000
N_PAD = 10240
D = 128
BN_EPS = 1e-5

E_TOTAL = 160000
E_PAD = 163840
N_TILES = 16
EPT = E_PAD // N_TILES
K = 32
CPT = EPT // K
SUPER = 8
NSUPER = CPT // SUPER
ROWS_E = E_PAD // K
ROWS_PER_TILE = 624
FIN_CHUNK = 16
SCAL_ROWS = 80
NODES_PER_TILE = N_PAD // N_TILES

TC_BLK = 1024


def _precompute_body(ent_ref, rel_ref, Wf_ref, w2_ref, b2_ref, bc_ref,
                     a0tab_ref, a1tab_ref, a2tab_ref, a0_ref, a1_ref, a2_ref):
    x = ent_ref[...]
    n = jnp.sqrt(jnp.sum(x * x, axis=1, keepdims=True))
    x = x * jnp.where(n > 1.0, 1.0 / (n + 1e-7), 1.0)
    y = rel_ref[...]
    m = jnp.sqrt(jnp.sum(y * y, axis=1, keepdims=True))
    y = y * jnp.where(m > 1.0, 1.0 / (m + 1e-7), 1.0)
    W = Wf_ref[...]
    dn = (((1,), (1,)), ((), ()))
    halfb = 0.5 * bc_ref[...]
    A0 = lax.dot_general(x, W[:, 0:D], dn, preferred_element_type=jnp.float32) + halfb
    A1 = lax.dot_general(x, W[:, D:2 * D], dn, preferred_element_type=jnp.float32) + halfb
    A2 = lax.dot_general(y, W[:, 2 * D:3 * D], dn, preferred_element_type=jnp.float32)
    w2 = w2_ref[...]
    b2 = b2_ref[0:1, 0:1]
    a0_ref[...] = lax.dot_general(w2, A0, dn, preferred_element_type=jnp.float32) + b2
    a1_ref[...] = lax.dot_general(w2, A1, dn, preferred_element_type=jnp.float32)
    a2_ref[...] = lax.dot_general(w2, A2, dn, preferred_element_type=jnp.float32)
    a0tab_ref[...] = A0
    a1tab_ref[...] = A1
    a2tab_ref[...] = A2


def _tc_precompute(ent_p, rel_p, Wf, w2, b2_arr, bc_arr):
    grid = (N_PAD // TC_BLK,)
    return pl.pallas_call(
        _precompute_body,
        grid=grid,
        in_specs=[
            pl.BlockSpec((TC_BLK, D), lambda i: (i, 0)),
            pl.BlockSpec((TC_BLK, D), lambda i: (i, 0)),
            pl.BlockSpec((D, 3 * D), lambda i: (0, 0)),
            pl.BlockSpec((1, D), lambda i: (0, 0)),
            pl.BlockSpec((1, D), lambda i: (0, 0)),
            pl.BlockSpec((1, D), lambda i: (0, 0)),
        ],
        out_specs=[
            pl.BlockSpec((TC_BLK, D), lambda i: (i, 0)),
            pl.BlockSpec((TC_BLK, D), lambda i: (i, 0)),
            pl.BlockSpec((TC_BLK, D), lambda i: (i, 0)),
            pl.BlockSpec((1, TC_BLK), lambda i: (0, i)),
            pl.BlockSpec((1, TC_BLK), lambda i: (0, i)),
            pl.BlockSpec((1, TC_BLK), lambda i: (0, i)),
        ],
        out_shape=[
            jax.ShapeDtypeStruct((N_PAD, D), jnp.float32),
            jax.ShapeDtypeStruct((N_PAD, D), jnp.float32),
            jax.ShapeDtypeStruct((N_PAD, D), jnp.float32),
            jax.ShapeDtypeStruct((1, N_PAD), jnp.float32),
            jax.ShapeDtypeStruct((1, N_PAD), jnp.float32),
            jax.ShapeDtypeStruct((1, N_PAD), jnp.float32),
        ],
    )(ent_p, rel_p, Wf, w2, b2_arr, bc_arr)


def _weights_body(a0_hbm, a1_hbm, a2_hbm, t0_hbm, t1_hbm, t2_hbm,
                  wf_hbm, wb_hbm, den8_hbm,
                  scal_acc, scal_acc2, t0f, t1f, t2f, wff, wbf, a0t, a1t, a2t,
                  ebs_l, ebs_l2, iden, zb16, d8b):
    cid = lax.axis_index("c")
    sid = lax.axis_index("s")
    is_ent = cid == 0
    flag = cid.astype(jnp.float32)
    base = sid * EPT

    pltpu.sync_copy(a0_hbm, a0t)
    pltpu.sync_copy(a1_hbm, a1t)
    pltpu.sync_copy(a2_hbm, a2t)
    pltpu.sync_copy(t0_hbm.at[pl.ds(base, EPT)], t0f.at[pl.ds(0, EPT)])
    pltpu.sync_copy(t1_hbm.at[pl.ds(base, EPT)], t1f.at[pl.ds(0, EPT)])
    pltpu.sync_copy(t2_hbm.at[pl.ds(base, EPT)], t2f.at[pl.ds(0, EPT)])

    iota16 = lax.iota(jnp.int32, 16)

    def zscal(g, c):
        iden[pl.ds(g * 16, 16)] = iota16 + g * 16
        return c
    lax.fori_loop(0, SCAL_ROWS // 16, zscal, 0)

    def zscal2(g, c):
        for j in range(D // 16):
            ebs_l[g, pl.ds(j * 16, 16)] = jnp.zeros((16,), jnp.float32)
            ebs_l2[g, pl.ds(j * 16, 16)] = jnp.zeros((16,), jnp.float32)
        return c
    lax.fori_loop(0, SCAL_ROWS, zscal2, 0)

    @pl.when(sid == 0)
    def _():
        def zr(i, c):
            for j in range(D // 16):
                zb16[i, pl.ds(j * 16, 16)] = jnp.zeros((16,), jnp.float32)
            return c
        lax.fori_loop(0, FIN_CHUNK, zr, 0)

        def zs(k, c):
            pltpu.sync_copy(zb16, scal_acc.at[pl.ds(k * FIN_CHUNK, FIN_CHUNK)])
            pltpu.sync_copy(zb16, scal_acc2.at[pl.ds(k * FIN_CHUNK, FIN_CHUNK)])
            return c
        lax.fori_loop(0, SCAL_ROWS // FIN_CHUNK, zs, 0)
    plsc.subcore_barrier()

    z16 = jnp.zeros((16,), jnp.int32)
    elim = jnp.full((16,), E_TOTAL, jnp.int32)

    def wstage(g, c):
        tv0 = t0f[pl.ds(g * 16, 16)]
        tv1 = t1f[pl.ds(g * 16, 16)]
        tv2 = t2f[pl.ds(g * 16, 16)]
        a0u = plsc.load_gather(a0t, [z16, tv0])
        a1u = plsc.load_gather(a1t, [z16, tv0])
        a0v = plsc.load_gather(a0t, [z16, tv1])
        a1v = plsc.load_gather(a1t, [z16, tv1])
        a2r = plsc.load_gather(a2t, [z16, tv2])
        zf = a0u + a1v + a2r
        zb = a0v + a1u - a2r
        gmask = ((iota16 + (base + g * 16)) < elim).astype(jnp.float32)
        wf = jnp.exp(jnp.minimum(-zf, -0.01 * zf)) * gmask
        wb = jnp.exp(jnp.minimum(-zb, -0.01 * zb)) * gmask
        wff[pl.ds(g * 16, 16)] = wf
        wbf[pl.ds(g * 16, 16)] = wb
        return c
    lax.fori_loop(0, EPT // 16, wstage, 0)

    @pl.when(is_ent)
    def _():
        pltpu.sync_copy(wff.at[pl.ds(0, EPT)], wf_hbm.at[pl.ds(base, EPT)])
        pltpu.sync_copy(wbf.at[pl.ds(0, EPT)], wb_hbm.at[pl.ds(base, EPT)])

    def acc_body(e, c):
        wf = wff[pl.ds(e, 16)][0]
        wb = wbf[pl.ds(e, 16)][0]
        t0s = t0f[pl.ds(e, 16)][0]
        t1s = t1f[pl.ds(e, 16)][0]
        t2s = t2f[pl.ds(e, 16)][0]
        m = jnp.where(base + e < E_TOTAL, jnp.float32(1.0), jnp.float32(0.0))
        na = t0s + (t2s - t0s) * cid
        ra = na >> 7
        ca = na & 112
        la = na & 15
        oh = (iota16 == la).astype(jnp.float32)
        da = wf + flag * (m - wf)
        ebs_l[ra, pl.ds(ca, 16)] = ebs_l[ra, pl.ds(ca, 16)] + oh * da

        @pl.when(is_ent)
        def _():
            rb = t1s >> 7
            cb2 = t1s & 112
            lb = t1s & 15
            ohb = (iota16 == lb).astype(jnp.float32) * wb
            ebs_l[rb, pl.ds(cb2, 16)] = ebs_l[rb, pl.ds(cb2, 16)] + ohb

        @pl.when(jnp.logical_not(is_ent))
        def _():
            ebs_l2[ra, pl.ds(ca, 16)] = ebs_l2[ra, pl.ds(ca, 16)] + oh * (wf + wb)
        return c
    lax.fori_loop(0, EPT, acc_body, 0)

    pltpu.sync_copy(ebs_l, scal_acc.at[iden], add=True)
    pltpu.sync_copy(ebs_l2, scal_acc2.at[iden], add=True)
    plsc.subcore_barrier()

    pltpu.sync_copy(scal_acc, ebs_l)
    pltpu.sync_copy(scal_acc2, ebs_l2)
    nbase = sid * NODES_PER_TILE
    f16 = iota16.astype(jnp.float32)
    mA0 = ((iota16 >> 2) == 0).astype(jnp.float32)
    mB0 = ((iota16 >> 2) == 1).astype(jnp.float32)
    mA1 = ((iota16 >> 2) == 2).astype(jnp.float32)
    mB1 = ((iota16 >> 2) == 3).astype(jnp.float32)
    del f16

    def expand(g, c):
        node0 = nbase + g * 16
        dr = node0 >> 7
        dc = node0 & 112
        dvA = ebs_l[dr, pl.ds(dc, 16)]
        dvB0 = ebs_l2[dr, pl.ds(dc, 16)]
        dvB = dvB0 + (1.0 - flag) * (dvA - dvB0)
        for h in range(8):
            pair = (mA0 * dvA[2 * h] + mB0 * dvB[2 * h]
                    + mA1 * dvA[2 * h + 1] + mB1 * dvB[2 * h + 1])
            d8b[pl.ds(g * 128 + h * 16, 16)] = pair
        return c
    lax.fori_loop(0, NODES_PER_TILE // 16, expand, 0)
    pltpu.sync_copy(d8b, den8_hbm.at[cid, pl.ds(nbase * 8, NODES_PER_TILE * 8)])


def _sc_weights(a0_t, a1_t, a2_t, t0, t1, t2):
    mesh = plsc.VectorSubcoreMesh(core_axis_name="c", subcore_axis_name="s")
    f = pl.kernel(
        _weights_body,
        out_type=(jax.ShapeDtypeStruct((E_PAD,), jnp.float32),
                  jax.ShapeDtypeStruct((E_PAD,), jnp.float32),
                  jax.ShapeDtypeStruct((2, N_PAD * 8), jnp.float32)),
        mesh=mesh,
        compiler_params=pltpu.CompilerParams(needs_layout_passes=False),
        scratch_types=[
            pltpu.VMEM_SHARED((SCAL_ROWS, D), jnp.float32),
            pltpu.VMEM_SHARED((SCAL_ROWS, D), jnp.float32),
            pltpu.VMEM((EPT + 16,), jnp.int32),
            pltpu.VMEM((EPT + 16,), jnp.int32),
            pltpu.VMEM((EPT + 16,), jnp.int32),
            pltpu.VMEM((EPT + 16,), jnp.float32),
            pltpu.VMEM((EPT + 16,), jnp.float32),
            pltpu.VMEM((1, N_PAD), jnp.float32),
            pltpu.VMEM((1, N_PAD), jnp.float32),
            pltpu.VMEM((1, N_PAD), jnp.float32),
            pltpu.VMEM((SCAL_ROWS, D), jnp.float32),
            pltpu.VMEM((SCAL_ROWS, D), jnp.float32),
            pltpu.VMEM((SCAL_ROWS,), jnp.int32),
            pltpu.VMEM((FIN_CHUNK, D), jnp.float32),
            pltpu.VMEM((NODES_PER_TILE * 8,), jnp.float32),
        ],
    )
    return f(a0_t, a1_t, a2_t, t0, t1, t2)


def _rows_body(T_hbm, gidx_hbm, sidx_hbm, wfr_hbm, wbr_hbm, acc3_hbm,
               acc, gst1, gst2, gst3, gst4, ssta, sstb, wst, wbst,
               b1a, b2a, b3a, b4a, b1b, b2b, b3b, b4b,
               sem_st, sem_g, sem_s):
    cid = lax.axis_index("c")
    sid = lax.axis_index("s")
    is_ent = cid == 0
    is_rel = jnp.logical_not(is_ent)
    flag = cid.astype(jnp.float32)

    n_fin = jnp.where(sid == N_TILES - 1, 40, 39)

    def zb4(i, c):
        for j in range(D // 16):
            b4a[i, pl.ds(j * 16, 16)] = jnp.zeros((16,), jnp.float32)
            b4b[i, pl.ds(j * 16, 16)] = jnp.zeros((16,), jnp.float32)
        return c
    lax.fori_loop(0, K, zb4, 0)

    def zrow(i, c):
        for j in range(D // 16):
            b1a[i, pl.ds(j * 16, 16)] = jnp.zeros((16,), jnp.float32)
        return c
    lax.fori_loop(0, FIN_CHUNK, zrow, 0)

    def zcopy(k, c):
        pltpu.sync_copy(b1a.at[pl.ds(0, FIN_CHUNK)],
                        acc.at[pl.ds(sid * ROWS_PER_TILE + k * FIN_CHUNK, FIN_CHUNK)])
        return c
    lax.fori_loop(0, n_fin, zcopy, 0)
    plsc.subcore_barrier()

    row_base = sid * CPT
    bufs = ((b1a, b2a, b3a, b4a), (b1b, b2b, b3b, b4b))

    def fire_gathers(k, par):
        c1 = pltpu.async_copy(T_hbm.at[gst1.at[k]], bufs[par][0], sem_g)
        c2 = pltpu.async_copy(T_hbm.at[gst2.at[k]], bufs[par][1], sem_g)
        c3 = pltpu.async_copy(T_hbm.at[gst3.at[k]], bufs[par][2], sem_g)
        cs = [c1, c2, c3]

        @pl.when(is_rel)
        def _():
            cs.append(pltpu.async_copy(T_hbm.at[gst4.at[k]], bufs[par][3], sem_g))
        return cs

    def wait_gathers(cs):
        cs[0].wait()
        cs[1].wait()
        cs[2].wait()

        @pl.when(is_rel)
        def _():
            cs[3].wait()

    def compute(k, par):
        c1, c2, c3, c4 = bufs[par]

        def grp(g, c):
            wfv = wst[k, pl.ds(g * 16, 16)]
            wbv = wbst[k, pl.ds(g * 16, 16)]
            for l in range(16):
                e = g * 16 + l
                wf = wfv[l]
                wb = wbv[l]
                for j in range(D // 16):
                    sl = pl.ds(j * 16, 16)
                    l1 = c1[e, sl]
                    l2 = c2[e, sl]
                    l3 = c3[e, sl]
                    l4 = c4[e, sl]
                    ent1 = wb * (l1 - l3)
                    rel1 = wf * (l1 + l2) - wb * (l3 + l4)
                    c2[e, sl] = wf * (l2 + l3)
                    c1[e, sl] = ent1 + flag * (rel1 - ent1)
            return c
        lax.fori_loop(0, K // 16, grp, 0)

    def fire_scatters(k, par):
        c1b, c2b = bufs[par][0], bufs[par][1]

        @pl.when(is_ent)
        def _():
            pltpu.async_copy(c2b, acc.at[ssta.at[k]], sem_s, add=True)
            pltpu.async_copy(c1b, acc.at[sstb.at[k]], sem_s, add=True)

        @pl.when(is_rel)
        def _():
            pltpu.async_copy(c1b, acc.at[ssta.at[k]], sem_s, add=True)

    def drain_scatters(par):
        @pl.when(is_ent)
        def _():
            pltpu.make_async_copy(T_hbm.at[pl.ds(0, K)], bufs[par][0], sem_s).wait()
            pltpu.make_async_copy(T_hbm.at[pl.ds(0, K)], bufs[par][1], sem_s).wait()

        @pl.when(is_rel)
        def _():
            pltpu.make_async_copy(T_hbm.at[pl.ds(0, K)], bufs[par][0], sem_s).wait()

    def drain_gathers(par):
        @pl.when(is_ent)
        def _():
            pltpu.make_async_copy(T_hbm.at[pl.ds(0, K)], bufs[par][0], sem_g).wait()
            pltpu.make_async_copy(T_hbm.at[pl.ds(0, K)], bufs[par][1], sem_g).wait()
            pltpu.make_async_copy(T_hbm.at[pl.ds(0, K)], bufs[par][2], sem_g).wait()

        @pl.when(is_rel)
        def _():
            pltpu.make_async_copy(T_hbm.at[pl.ds(0, K)], bufs[par][0], sem_g).wait()
            pltpu.make_async_copy(T_hbm.at[pl.ds(0, K)], bufs[par][1], sem_g).wait()
            pltpu.make_async_copy(T_hbm.at[pl.ds(0, K)], bufs[par][2], sem_g).wait()
            pltpu.make_async_copy(T_hbm.at[pl.ds(0, K)], bufs[par][3], sem_g).wait()

    def fire_gathers_nh(k, par):
        pltpu.async_copy(T_hbm.at[gst1.at[k]], bufs[par][0], sem_g)
        pltpu.async_copy(T_hbm.at[gst2.at[k]], bufs[par][1], sem_g)
        pltpu.async_copy(T_hbm.at[gst3.at[k]], bufs[par][2], sem_g)

        @pl.when(is_rel)
        def _():
            pltpu.async_copy(T_hbm.at[gst4.at[k]], bufs[par][3], sem_g)

    NPAIRS = CPT // 2
    HSUPER = SUPER // 2

    def pair_body(p, carry):
        @pl.when(p % HSUPER == 0)
        def _():
            rb = row_base + (p // HSUPER) * SUPER
            s = [pltpu.async_copy(gidx_hbm.at[cid, 0, pl.ds(rb, SUPER)], gst1, sem_st),
                 pltpu.async_copy(gidx_hbm.at[cid, 1, pl.ds(rb, SUPER)], gst2, sem_st),
                 pltpu.async_copy(gidx_hbm.at[cid, 2, pl.ds(rb, SUPER)], gst3, sem_st),
                 pltpu.async_copy(gidx_hbm.at[cid, 3, pl.ds(rb, SUPER)], gst4, sem_st),
                 pltpu.async_copy(sidx_hbm.at[cid, 0, pl.ds(rb, SUPER)], ssta, sem_st),
                 pltpu.async_copy(sidx_hbm.at[cid, 1, pl.ds(rb, SUPER)], sstb, sem_st),
                 pltpu.async_copy(wfr_hbm.at[cid, pl.ds(rb, SUPER)], wst, sem_st),
                 pltpu.async_copy(wbr_hbm.at[cid, pl.ds(rb, SUPER)], wbst, sem_st)]
            for c in s:
                c.wait()
            fire_gathers_nh((2 * p) % SUPER, 0)

        kA = (2 * p) % SUPER
        kB = kA + 1

        @pl.when(p != 0)
        def _():
            drain_scatters(1)

        gb = fire_gathers(kB, 1)

        drain_gathers(0)
        compute(kA, 0)
        fire_scatters(kA, 0)

        wait_gathers(gb)
        compute(kB, 1)
        fire_scatters(kB, 1)

        drain_scatters(0)

        @pl.when(jnp.logical_and((p + 1) % HSUPER != 0, p + 1 < NPAIRS))
        def _():
            fire_gathers_nh((2 * p + 2) % SUPER, 0)
        return carry

    lax.fori_loop(0, NPAIRS, pair_body, 0)
    drain_scatters(1)
    plsc.subcore_barrier()

    abase = sid * ROWS_PER_TILE
    pltpu.sync_copy(acc.at[pl.ds(abase, ROWS_PER_TILE)],
                    acc3_hbm.at[cid, pl.ds(abase, ROWS_PER_TILE)])

    @pl.when(sid == N_TILES - 1)
    def _():
        pltpu.sync_copy(acc.at[pl.ds(N_ENT - FIN_CHUNK, FIN_CHUNK)],
                        acc3_hbm.at[cid, pl.ds(N_ENT - FIN_CHUNK, FIN_CHUNK)])


def _sc_rows(T, gidx, sidx, wfr, wbr):
    mesh = plsc.VectorSubcoreMesh(core_axis_name="c", subcore_axis_name="s")
    f = pl.kernel(
        _rows_body,
        out_type=jax.ShapeDtypeStruct((2, N_ENT, D), jnp.float32),
        mesh=mesh,
        compiler_params=pltpu.CompilerParams(needs_layout_passes=False),
        scratch_types=[
            pltpu.VMEM_SHARED((N_ENT, D), jnp.float32),
            pltpu.VMEM((SUPER, K), jnp.int32),
            pltpu.VMEM((SUPER, K), jnp.int32),
            pltpu.VMEM((SUPER, K), jnp.int32),
            pltpu.VMEM((SUPER, K), jnp.int32),
            pltpu.VMEM((SUPER, K), jnp.int32),
            pltpu.VMEM((SUPER, K), jnp.int32),
            pltpu.VMEM((SUPER, K), jnp.float32),
            pltpu.VMEM((SUPER, K), jnp.float32),
            pltpu.VMEM((K, D), jnp.float32),
            pltpu.VMEM((K, D), jnp.float32),
            pltpu.VMEM((K, D), jnp.float32),
            pltpu.VMEM((K, D), jnp.float32),
            pltpu.VMEM((K, D), jnp.float32),
            pltpu.VMEM((K, D), jnp.float32),
            pltpu.VMEM((K, D), jnp.float32),
            pltpu.VMEM((K, D), jnp.float32),
            pltpu.SemaphoreType.DMA,
            pltpu.SemaphoreType.DMA,
            pltpu.SemaphoreType.DMA,
        ],
    )
    return f(T, gidx, sidx, wfr, wbr)


def _finalize_body(acc_ref, den_ref, diag_ref, out_ref):
    c = pl.program_id(0)
    num = acc_ref[0]
    den = den_ref[0][:, 0:1]
    scale = den_ref[0][:, 4:5]
    num = num + diag_ref[0] * scale
    den_e = jnp.where(den == 0.0, jnp.float32(1e-12), den)
    den_r = jnp.maximum(den, 1.0)
    den = jnp.where(c == 0, den_e, den_r)
    h = num / den
    out_ref[0] = jnp.where(h > 0.0, h, jnp.exp(h) - 1.0)


TC_BLK_F = 1000


def _tc_finalize(acc3, den8, diag):
    grid = (2, N_ENT // TC_BLK_F)
    return pl.pallas_call(
        _finalize_body,
        grid=grid,
        in_specs=[
            pl.BlockSpec((1, TC_BLK_F, D), lambda c, i: (c, i, 0)),
            pl.BlockSpec((1, TC_BLK_F, 8), lambda c, i: (c, i, 0)),
            pl.BlockSpec((1, TC_BLK_F, D), lambda c, i: (c, i, 0)),
        ],
        out_specs=pl.BlockSpec((1, TC_BLK_F, D), lambda c, i: (c, i, 0)),
        out_shape=jax.ShapeDtypeStruct((2, N_ENT, D), jnp.float32),
    )(acc3, den8, diag)


def kernel(triplets, ent_w, rel_w, W_a, b_a, W_a2, b_a2, g0, be0, g1, be1):
    inv = 1.0 / jnp.sqrt(jnp.float32(1.0 + BN_EPS))
    s0 = g0 * inv
    s1 = g1 * inv
    Wf = (s1[:, None] * W_a) * s0[None, :]
    bias_c = s1 * (W_a @ be0 + b_a) + be1
    b2_arr = jnp.full((1, D), b_a2[0], jnp.float32)
    bc_arr = bias_c.reshape(1, D)

    ent_p = jnp.pad(ent_w, ((0, N_PAD - N_ENT), (0, 0)))
    rel_p = jnp.pad(rel_w, ((0, N_PAD - N_REL), (0, 0)))

    a0tab, a1tab, a2tab, a0_t, a1_t, a2_t = _tc_precompute(
        ent_p, rel_p, Wf, W_a2, b2_arr, bc_arr)

    pad = E_PAD - E_TOTAL
    t0 = jnp.pad(triplets[:, 0], (0, pad))
    t1 = jnp.pad(triplets[:, 1], (0, pad))
    t2 = jnp.pad(triplets[:, 2], (0, pad))

    wf, wb, den8f = _sc_weights(a0_t, a1_t, a2_t, t0, t1, t2)
    den8 = den8f.reshape(2, N_PAD, 8)

    T = jnp.concatenate([a0tab, a1tab, a2tab], axis=0)

    gidx = jnp.stack([
        jnp.stack([t0 + N_PAD, t1 + N_PAD, t2 + 2 * N_PAD, t0 + N_PAD]),
        jnp.stack([t0, t1 + N_PAD, t1, t0 + N_PAD]),
    ]).reshape(2, 4, ROWS_E, K)
    sidx = jnp.stack([
        jnp.stack([t0, t1]),
        jnp.stack([t2, t2]),
    ]).reshape(2, 2, ROWS_E, K)
    wfr = jnp.broadcast_to(wf.reshape(1, ROWS_E, K), (2, ROWS_E, K))
    wbr = jnp.broadcast_to(wb.reshape(1, ROWS_E, K), (2, ROWS_E, K))

    acc3 = _sc_rows(T, gidx, sidx, wfr, wbr)

    diag = jnp.stack([a0tab[:N_ENT], a2tab[:N_ENT]])

    h3 = _tc_finalize(acc3, den8[:, :N_ENT, :], diag)
    return h3[0], h3[1]
